# baseline (speedup 1.0000x reference)
import functools

import jax
import jax.numpy as jnp
from jax import lax
from jax.experimental import pallas as pl
from jax.experimental.pallas import tpu as pltpu
from jax.experimental.pallas import tpu_sc as plsc

_TOP_K = 8
_L = 16
_NW = 32


def _dense_body(x_ref, wt_ref, b_ref, p_ref, k_ref, *, tm, e):
    logits = jnp.dot(x_ref[...], wt_ref[...], preferred_element_type=jnp.float32)
    m = jnp.max(logits, axis=-1, keepdims=True)
    ex = jnp.exp(logits - m)
    p_ref[...] = ex / jnp.sum(ex, axis=-1, keepdims=True)

    work = logits + b_ref[...]
    ki = lax.bitcast_convert_type(work, jnp.int32)
    ki = ki ^ ((ki >> 31) & jnp.int32(0x7FFFFFFF))
    cols = lax.broadcasted_iota(jnp.int32, (tm, e), 1)
    ki = (ki & jnp.int32(~0x3F)) | (jnp.int32(e - 1) - cols)
    ki = ki ^ ((ki >> 31) & jnp.int32(0x7FFFFFFF))
    keys = lax.bitcast_convert_type(ki, jnp.float32)
    k_ref[0] = keys.T


def _dense(x, wt, bias2d):
    tokens, dim = x.shape
    e = wt.shape[1]
    tm = tokens // _NW
    n = _NW
    return pl.pallas_call(
        functools.partial(_dense_body, tm=tm, e=e),
        grid=(n,),
        in_specs=[
            pl.BlockSpec((tm, dim), lambda i: (i, 0)),
            pl.BlockSpec((dim, e), lambda i: (0, 0)),
            pl.BlockSpec((1, e), lambda i: (0, 0)),
        ],
        out_specs=(
            pl.BlockSpec((tm, e), lambda i: (i, 0)),
            pl.BlockSpec((1, e, tm), lambda i: (i, 0, 0)),
        ),
        out_shape=(
            jax.ShapeDtypeStruct((tokens, e), jnp.float32),
            jax.ShapeDtypeStruct((_NW, e, tm), jnp.float32),
        ),
    )(x, wt, bias2d)


def _make_router(tokens, e):
    tpw = tokens // _NW
    mesh = plsc.VectorSubcoreMesh(core_axis_name="c", subcore_axis_name="s")

    @functools.partial(
        pl.kernel,
        mesh=mesh,
        out_type=(
            jax.ShapeDtypeStruct((_NW, _TOP_K, tpw), jnp.int32),
            jax.ShapeDtypeStruct((_NW, _TOP_K, tpw), jnp.float32),
        ),
        scratch_types=[
            pltpu.VMEM((e, tpw), jnp.float32),
            pltpu.VMEM((_TOP_K, tpw), jnp.int32),
            pltpu.VMEM((_TOP_K, tpw), jnp.float32),
        ],
    )
    def route(key_hbm, idx_hbm, w_hbm, kbuf, ibuf, wbuf):
        wid = lax.axis_index("s") * 2 + lax.axis_index("c")
        pltpu.sync_copy(key_hbm.at[wid], kbuf)

        neg_inf = jnp.full((_L,), -jnp.inf, jnp.float32)

        def group(g, carry):
            col = g * _L
            t = [neg_inf] * _TOP_K
            for exp_i in range(e):
                c = kbuf[exp_i, pl.ds(col, _L)]
                for m in range(_TOP_K):
                    hi = jnp.maximum(t[m], c)
                    c = jnp.minimum(t[m], c)
                    t[m] = hi
            ksum = jnp.zeros((_L,), jnp.float32)
            ews = []
            for m in range(_TOP_K):
                tki = lax.bitcast_convert_type(t[m], jnp.int32)
                low = tki & jnp.int32(0x3F)
                ibuf[m, pl.ds(col, _L)] = jnp.where(
                    tki < 0, low, jnp.int32(e - 1) - low)
                ew = jnp.exp(t[m] - t[0])
                ews.append(ew)
                ksum = ksum + ew
            for m in range(_TOP_K):
                wbuf[m, pl.ds(col, _L)] = ews[m] / ksum
            return carry

        lax.fori_loop(0, tpw // _L, group, 0)

        pltpu.sync_copy(ibuf, idx_hbm.at[wid])
        pltpu.sync_copy(wbuf, w_hbm.at[wid])

    return route


def _load_body(idx_ref, load_ref, *, e):
    idx = idx_ref[...]
    lane = lax.broadcasted_iota(jnp.int32, (1, e), 1)
    acc = jnp.zeros((1, e), jnp.float32)
    for exp_i in range(e):
        cnt = jnp.sum(jnp.where(idx == exp_i, 1.0, 0.0))
        acc = jnp.where(lane == exp_i, cnt, acc)
    load_ref[...] = acc


def _loads(indices, e):
    tokens, k = indices.shape
    rows = tokens * k // 128
    flat = indices.reshape(rows, 128)
    return pl.pallas_call(
        functools.partial(_load_body, e=e),
        grid=(1,),
        in_specs=[pl.BlockSpec((rows, 128), lambda i: (0, 0))],
        out_specs=pl.BlockSpec((1, e), lambda i: (0, 0)),
        out_shape=jax.ShapeDtypeStruct((1, e), jnp.float32),
    )(flat)


def kernel(x, W, router_bias):
    tokens, dim = x.shape
    e = W.shape[0]
    wt = W.T
    bias2d = router_bias.reshape(1, e)

    probs, keys_t = _dense(x, wt, bias2d)
    idx_t, w_t = _make_router(tokens, e)(keys_t)
    indices = idx_t.transpose(0, 2, 1).reshape(tokens, _TOP_K)
    weights = w_t.transpose(0, 2, 1).reshape(tokens, _TOP_K)
    load = _loads(indices, e)
    return indices, weights.astype(x.dtype), probs, load.reshape(e)


# speedup vs baseline: 1.0013x; 1.0013x over previous
import functools

import jax
import jax.numpy as jnp
from jax import lax
from jax.experimental import pallas as pl
from jax.experimental.pallas import tpu as pltpu

_TOP_K = 8


def _router_body(x_ref, wt_ref, b_ref, idx_ref, w_ref, p_ref, load_ref, lbuf,
                 *, tm, e, n):
    i = pl.program_id(0)
    slot = lax.rem(i, 2)

    lbuf[slot] = jnp.dot(x_ref[...], wt_ref[...],
                         preferred_element_type=jnp.float32)

    logits = lbuf[1 - slot]

    m = jnp.max(logits, axis=-1, keepdims=True)
    ex = jnp.exp(logits - m)
    probs = ex / jnp.sum(ex, axis=-1, keepdims=True)
    p_ref[...] = probs

    work = logits + b_ref[...]
    ki = lax.bitcast_convert_type(work, jnp.int32)
    ki = ki ^ ((ki >> 31) & jnp.int32(0x7FFFFFFF))
    cols = lax.broadcasted_iota(jnp.int32, (tm, e), 1)
    ki = (ki & jnp.int32(~0x3F)) | (jnp.int32(e - 1) - cols)
    ki = ki ^ ((ki >> 31) & jnp.int32(0x7FFFFFFF))
    key = lax.bitcast_convert_type(ki, jnp.float32)

    sel = jnp.zeros((tm, e), dtype=jnp.bool_)
    key_cols = []
    neg_inf = jnp.float32(-jnp.inf)
    for _ in range(_TOP_K):
        mk = jnp.max(key, axis=-1, keepdims=True)
        onehot = key == mk
        key_cols.append(mk)
        sel = sel | onehot
        key = jnp.where(onehot, neg_inf, key)

    topk = jnp.concatenate(key_cols, axis=-1)
    tki = lax.bitcast_convert_type(topk, jnp.int32)
    low = tki & jnp.int32(0x3F)
    indices = jnp.where(tki < 0, low, jnp.int32(e - 1) - low)
    ew = jnp.exp(topk - m)
    weights = ew / jnp.clip(jnp.sum(ew, axis=-1, keepdims=True), 1e-9, None)
    idx_ref[...] = indices
    w_ref[...] = weights

    @pl.when(i == 1)
    def _():
        load_ref[...] = jnp.zeros_like(load_ref)

    @pl.when(i >= 1)
    def _():
        load_ref[...] += jnp.sum(sel.astype(jnp.float32), axis=0, keepdims=True)


def kernel(x, W, router_bias):
    tokens, dim = x.shape
    e = W.shape[0]
    tm = min(512, tokens)
    n = tokens // tm

    wt = W.T
    bias2d = router_bias.reshape(1, e)

    out_shapes = (
        jax.ShapeDtypeStruct((tokens, _TOP_K), jnp.int32),
        jax.ShapeDtypeStruct((tokens, _TOP_K), jnp.float32),
        jax.ShapeDtypeStruct((tokens, e), jnp.float32),
        jax.ShapeDtypeStruct((1, e), jnp.float32),
    )
    indices, weights, probs, load = pl.pallas_call(
        functools.partial(_router_body, tm=tm, e=e, n=n),
        grid=(n + 1,),
        in_specs=[
            pl.BlockSpec((tm, dim), lambda i: (jnp.minimum(i, n - 1), 0)),
            pl.BlockSpec((dim, e), lambda i: (0, 0)),
            pl.BlockSpec((1, e), lambda i: (0, 0)),
        ],
        out_specs=(
            pl.BlockSpec((tm, _TOP_K), lambda i: (jnp.maximum(i - 1, 0), 0)),
            pl.BlockSpec((tm, _TOP_K), lambda i: (jnp.maximum(i - 1, 0), 0)),
            pl.BlockSpec((tm, e), lambda i: (jnp.maximum(i - 1, 0), 0)),
            pl.BlockSpec((1, e), lambda i: (0, 0)),
        ),
        out_shape=out_shapes,
        scratch_shapes=[pltpu.VMEM((2, tm, e), jnp.float32)],
    )(x, wt, bias2d)

    return indices, weights.astype(x.dtype), probs, load.reshape(e)
